# Initial kernel scaffold; baseline (speedup 1.0000x reference)
import functools

import jax
import jax.numpy as jnp
from jax import lax
from jax.experimental import pallas as pl
from jax.experimental.pallas import tpu as pltpu
from jax.experimental.pallas import tpu_sc as plsc

N_NODES = 10000
D = 128
N_EDGES = 320000

NC = 2
NS = 16
NW = NC * NS
EDGES_PER_W = N_EDGES // NW
CHUNK = 80
STEPS = EDGES_PER_W // CHUNK
ROWS_PER_TILE = N_NODES // NS
DEG_W = 16


def _deg_sc_body(dst_hbm, zeros_hbm, cnt_hbm, didx_v, ones_v, deg_sh):
    c = lax.axis_index("c")
    s = lax.axis_index("s")
    wid = c * NS + s

    one16 = jnp.full((16,), 1.0, dtype=jnp.float32)
    def fill(i, _):
        ones_v[i, pl.ds(0, 16)] = one16
        return 0
    lax.fori_loop(0, CHUNK, fill, 0)

    pltpu.sync_copy(zeros_hbm.at[pl.ds(s * ROWS_PER_TILE, ROWS_PER_TILE)],
                    deg_sh.at[pl.ds(s * ROWS_PER_TILE, ROWS_PER_TILE)])
    plsc.subcore_barrier()

    base = wid * EDGES_PER_W
    def step(i, _):
        pltpu.sync_copy(dst_hbm.at[pl.ds(base + i * CHUNK, CHUNK)], didx_v)
        pltpu.sync_copy(ones_v, deg_sh.at[didx_v], add=True)
        return 0
    lax.fori_loop(0, STEPS, step, 0)

    plsc.subcore_barrier()
    pltpu.sync_copy(deg_sh.at[pl.ds(s * ROWS_PER_TILE, ROWS_PER_TILE)],
                    cnt_hbm.at[c, pl.ds(s * ROWS_PER_TILE, ROWS_PER_TILE)])


def _scatter_sc_body(y_hbm, src_hbm, dst_hbm, zeros_hbm, z_hbm,
                     sidx_v, didx_v, rows_v, z_sh, sem):
    c = lax.axis_index("c")
    s = lax.axis_index("s")
    wid = c * NS + s

    rlo = s * ROWS_PER_TILE
    @pl.when(c == 0)
    def _():
        pltpu.sync_copy(y_hbm.at[pl.ds(rlo, ROWS_PER_TILE)],
                        z_sh.at[pl.ds(rlo, ROWS_PER_TILE)])
    @pl.when(c != 0)
    def _():
        pltpu.sync_copy(zeros_hbm.at[pl.ds(rlo, ROWS_PER_TILE)],
                        z_sh.at[pl.ds(rlo, ROWS_PER_TILE)])
    plsc.subcore_barrier()

    base = wid * EDGES_PER_W
    def step(i, _):
        pltpu.sync_copy(src_hbm.at[pl.ds(base + i * CHUNK, CHUNK)], sidx_v)
        pltpu.sync_copy(dst_hbm.at[pl.ds(base + i * CHUNK, CHUNK)], didx_v)
        pltpu.async_copy(y_hbm.at[sidx_v], rows_v, sem).wait()
        pltpu.sync_copy(rows_v, z_sh.at[didx_v], add=True)
        return 0
    lax.fori_loop(0, STEPS, step, 0)

    plsc.subcore_barrier()
    pltpu.sync_copy(z_sh.at[pl.ds(rlo, ROWS_PER_TILE)],
                    z_hbm.at[c, pl.ds(rlo, ROWS_PER_TILE)])


def _scale_tc_body(x_ref, cnt_ref, y_ref, s_ref):
    cnt = cnt_ref[0, :, 0:1] + cnt_ref[1, :, 0:1]
    s = lax.rsqrt(cnt + 1.0)
    s_ref[...] = s
    y_ref[...] = x_ref[...] * s


def _matmul_tc_body(z_ref, s_ref, wm_ref, bm_ref, wl_ref, bl_ref,
                    mu_ref, ls_ref):
    agg = (z_ref[0] + z_ref[1]) * s_ref[...]
    mu_ref[...] = jnp.dot(agg, wm_ref[...],
                          preferred_element_type=jnp.float32,
                          precision=lax.Precision.HIGHEST) + bm_ref[...]
    ls_ref[...] = jnp.dot(agg, wl_ref[...],
                          preferred_element_type=jnp.float32,
                          precision=lax.Precision.HIGHEST) + bl_ref[...]


_SC_MESH = plsc.VectorSubcoreMesh(core_axis_name="c", subcore_axis_name="s")

_deg_call = pl.kernel(
    _deg_sc_body,
    out_type=jax.ShapeDtypeStruct((NC, N_NODES, DEG_W), jnp.float32),
    mesh=_SC_MESH,
    scratch_types=[
        pltpu.VMEM((CHUNK,), jnp.int32),
        pltpu.VMEM((CHUNK, DEG_W), jnp.float32),
        pltpu.VMEM_SHARED((N_NODES, DEG_W), jnp.float32),
    ],
)

_scatter_call = pl.kernel(
    _scatter_sc_body,
    out_type=jax.ShapeDtypeStruct((NC, N_NODES, D), jnp.float32),
    mesh=_SC_MESH,
    scratch_types=[
        pltpu.VMEM((CHUNK,), jnp.int32),
        pltpu.VMEM((CHUNK,), jnp.int32),
        pltpu.VMEM((CHUNK, D), jnp.float32),
        pltpu.VMEM_SHARED((N_NODES, D), jnp.float32),
        pltpu.SemaphoreType.DMA,
    ],
)


@jax.jit
def kernel(x, edge_index, W_mu, b_mu, W_logstd, b_logstd):
    src = edge_index[0].astype(jnp.int32)
    dst = edge_index[1].astype(jnp.int32)
    zeros16 = jnp.zeros((N_NODES, DEG_W), jnp.float32)
    zeros_nd = jnp.zeros((N_NODES, D), jnp.float32)

    cnt = _deg_call(dst, zeros16)

    y, s = pl.pallas_call(
        _scale_tc_body,
        out_shape=(
            jax.ShapeDtypeStruct((N_NODES, D), jnp.float32),
            jax.ShapeDtypeStruct((N_NODES, 1), jnp.float32),
        ),
    )(x, cnt)

    z = _scatter_call(y, src, dst, zeros_nd)

    mu, logstd = pl.pallas_call(
        _matmul_tc_body,
        out_shape=(
            jax.ShapeDtypeStruct((N_NODES, D), jnp.float32),
            jax.ShapeDtypeStruct((N_NODES, D), jnp.float32),
        ),
    )(z, s, W_mu, b_mu.reshape(1, D), W_logstd, b_logstd.reshape(1, D))

    return (mu, logstd)


# speedup vs baseline: 21.7400x; 21.7400x over previous
import jax
import jax.numpy as jnp
from jax import lax
from jax.experimental import pallas as pl
from jax.experimental.pallas import tpu as pltpu
from jax.experimental.pallas import tpu_sc as plsc

N_NODES = 10000
N_PAD = 10240
D = 128
N_EDGES = 320000

NC = 2
NS = 16
NW = NC * NS
EDGES_PER_W = N_EDGES // NW
CHUNK = 80
STEPS = EDGES_PER_W // CHUNK
ROWS_PER_TILE = N_PAD // NS
DEG_W = 16


def _deg_sc_body(dst_hbm, zeros_hbm, cnt_hbm, didx_v, ones_v, deg_sh):
    c = lax.axis_index("c")
    s = lax.axis_index("s")
    wid = c * NS + s

    one16 = jnp.full((16,), 1.0, dtype=jnp.float32)
    def fill(i, _):
        ones_v[i, pl.ds(0, 16)] = one16
        return 0
    lax.fori_loop(0, CHUNK, fill, 0)

    rlo = s * ROWS_PER_TILE
    pltpu.sync_copy(zeros_hbm.at[pl.ds(rlo, ROWS_PER_TILE)],
                    deg_sh.at[pl.ds(rlo, ROWS_PER_TILE)])
    plsc.subcore_barrier()

    base = wid * EDGES_PER_W
    def step(i, _):
        pltpu.sync_copy(dst_hbm.at[pl.ds(base + i * CHUNK, CHUNK)], didx_v)
        pltpu.sync_copy(ones_v, deg_sh.at[didx_v], add=True)
        return 0
    lax.fori_loop(0, STEPS, step, 0)

    plsc.subcore_barrier()
    pltpu.sync_copy(deg_sh.at[pl.ds(rlo, ROWS_PER_TILE)],
                    cnt_hbm.at[c, pl.ds(rlo, ROWS_PER_TILE)])


def _scatter_sc_body(y_hbm, src_hbm, dst_hbm, zeros_hbm, z_hbm,
                     sidx_v, didx_v, rows_v, z_sh, sem):
    c = lax.axis_index("c")
    s = lax.axis_index("s")
    wid = c * NS + s

    rlo = s * ROWS_PER_TILE
    @pl.when(c == 0)
    def _():
        pltpu.sync_copy(y_hbm.at[pl.ds(rlo, ROWS_PER_TILE)],
                        z_sh.at[pl.ds(rlo, ROWS_PER_TILE)])
    @pl.when(c != 0)
    def _():
        pltpu.sync_copy(zeros_hbm.at[pl.ds(rlo, ROWS_PER_TILE)],
                        z_sh.at[pl.ds(rlo, ROWS_PER_TILE)])
    plsc.subcore_barrier()

    base = wid * EDGES_PER_W
    def step(i, _):
        pltpu.sync_copy(src_hbm.at[pl.ds(base + i * CHUNK, CHUNK)], sidx_v)
        pltpu.sync_copy(dst_hbm.at[pl.ds(base + i * CHUNK, CHUNK)], didx_v)
        pltpu.async_copy(y_hbm.at[sidx_v], rows_v, sem).wait()
        pltpu.sync_copy(rows_v, z_sh.at[didx_v], add=True)
        return 0
    lax.fori_loop(0, STEPS, step, 0)

    plsc.subcore_barrier()
    pltpu.sync_copy(z_sh.at[pl.ds(rlo, ROWS_PER_TILE)],
                    z_hbm.at[c, pl.ds(rlo, ROWS_PER_TILE)])


def _scale_tc_body(x_ref, cnt_ref, y_ref, s_ref):
    cnt = cnt_ref[0, 0:N_NODES, 0:1] + cnt_ref[1, 0:N_NODES, 0:1]
    s = lax.rsqrt(cnt + 1.0)
    s_ref[...] = s
    y_ref[0:N_NODES, :] = x_ref[...] * s
    y_ref[N_NODES:N_PAD, :] = jnp.zeros((N_PAD - N_NODES, D), jnp.float32)


def _matmul_tc_body(z_ref, s_ref, wm_ref, bm_ref, wl_ref, bl_ref,
                    mu_ref, ls_ref):
    agg = (z_ref[0, 0:N_NODES, :] + z_ref[1, 0:N_NODES, :]) * s_ref[...]
    mu_ref[...] = jnp.dot(agg, wm_ref[...],
                          preferred_element_type=jnp.float32,
                          precision=lax.Precision.HIGHEST) + bm_ref[...]
    ls_ref[...] = jnp.dot(agg, wl_ref[...],
                          preferred_element_type=jnp.float32,
                          precision=lax.Precision.HIGHEST) + bl_ref[...]


_SC_MESH = plsc.VectorSubcoreMesh(core_axis_name="c", subcore_axis_name="s")

_deg_call = pl.kernel(
    _deg_sc_body,
    out_type=jax.ShapeDtypeStruct((NC, N_PAD, DEG_W), jnp.float32),
    mesh=_SC_MESH,
    scratch_types=[
        pltpu.VMEM((CHUNK,), jnp.int32),
        pltpu.VMEM((CHUNK, DEG_W), jnp.float32),
        pltpu.VMEM_SHARED((N_PAD, DEG_W), jnp.float32),
    ],
)

_scatter_call = pl.kernel(
    _scatter_sc_body,
    out_type=jax.ShapeDtypeStruct((NC, N_PAD, D), jnp.float32),
    mesh=_SC_MESH,
    scratch_types=[
        pltpu.VMEM((CHUNK,), jnp.int32),
        pltpu.VMEM((CHUNK,), jnp.int32),
        pltpu.VMEM((CHUNK, D), jnp.float32),
        pltpu.VMEM_SHARED((N_PAD, D), jnp.float32),
        pltpu.SemaphoreType.DMA,
    ],
)


@jax.jit
def kernel(x, edge_index, W_mu, b_mu, W_logstd, b_logstd):
    src = edge_index[0].astype(jnp.int32)
    dst = edge_index[1].astype(jnp.int32)
    zeros16 = jnp.zeros((N_PAD, DEG_W), jnp.float32)
    zeros_nd = jnp.zeros((N_PAD, D), jnp.float32)

    cnt = _deg_call(dst, zeros16)

    y, s = pl.pallas_call(
        _scale_tc_body,
        out_shape=(
            jax.ShapeDtypeStruct((N_PAD, D), jnp.float32),
            jax.ShapeDtypeStruct((N_NODES, 1), jnp.float32),
        ),
    )(x, cnt)

    z = _scatter_call(y, src, dst, zeros_nd)

    mu, logstd = pl.pallas_call(
        _matmul_tc_body,
        out_shape=(
            jax.ShapeDtypeStruct((N_NODES, D), jnp.float32),
            jax.ShapeDtypeStruct((N_NODES, D), jnp.float32),
        ),
    )(z, s, W_mu, b_mu.reshape(1, D), W_logstd, b_logstd.reshape(1, D))

    return (mu, logstd)
